# Initial kernel scaffold; baseline (speedup 1.0000x reference)
import jax
import jax.numpy as jnp
from jax.experimental import pallas as pl


def kernel(z_e, embed):
    raise NotImplementedError("write your pallas kernel here")


# speedup vs baseline: 2.3805x; 2.3805x over previous
import jax
import jax.numpy as jnp
from jax.experimental import pallas as pl


def _vq_body(ze_ref, emb_ref, embT_ref, zq_ref, idx_ref, loss_ref):
    ze = ze_ref[0]
    emb = emb_ref[...]
    embT = embT_ref[...]
    C = emb.shape[0]
    P = ze.shape[1]
    scores = jnp.dot(emb, ze, preferred_element_type=jnp.float32)
    enorm = jnp.sum(emb * emb, axis=1, keepdims=True)
    fnorm = jnp.sum(ze * ze, axis=0, keepdims=True)
    dist = (fnorm + enorm) - 2.0 * scores
    idx = jnp.argmin(dist, axis=0)
    onehot = (jax.lax.broadcasted_iota(jnp.int32, (C, P), 0)
              == idx[None, :]).astype(jnp.float32)
    zq = jnp.dot(embT, onehot, preferred_element_type=jnp.float32)
    zq_ref[0] = zq
    idx_ref[0] = idx.reshape(1, P).astype(jnp.int32)
    diff = ze - zq
    loss_ref[...] = jnp.sum(diff * diff).reshape(1, 1, 1)


def kernel(z_e, embed):
    B, D, H, W = z_e.shape
    P = H * W
    C = embed.shape[0]
    ze = z_e.reshape(B, D, P)
    embT = embed.T

    zq, idx, partial = pl.pallas_call(
        _vq_body,
        grid=(B,),
        in_specs=[
            pl.BlockSpec((1, D, P), lambda b: (b, 0, 0)),
            pl.BlockSpec((C, D), lambda b: (0, 0)),
            pl.BlockSpec((D, C), lambda b: (0, 0)),
        ],
        out_specs=[
            pl.BlockSpec((1, D, P), lambda b: (b, 0, 0)),
            pl.BlockSpec((1, 1, P), lambda b: (b, 0, 0)),
            pl.BlockSpec((1, 1, 1), lambda b: (b, 0, 0)),
        ],
        out_shape=[
            jax.ShapeDtypeStruct((B, D, P), jnp.float32),
            jax.ShapeDtypeStruct((B, 1, P), jnp.int32),
            jax.ShapeDtypeStruct((B, 1, 1), jnp.float32),
        ],
    )(ze, embed, embT)

    z_q_st = zq.reshape(B, D, H, W)
    commitment_loss = jnp.sum(partial) / (B * P * D)
    indices_out = idx.reshape(B, H, W)
    return (z_q_st, commitment_loss, indices_out)


# speedup vs baseline: 2.4372x; 1.0238x over previous
import jax
import jax.numpy as jnp
from jax.experimental import pallas as pl
from jax.experimental.pallas import tpu as pltpu


def _vq_body(ze_ref, emb_ref, zq_ref, idx_ref, loss_ref, enorm_ref):
    b = pl.program_id(0)
    nb = pl.num_programs(0)
    ze = ze_ref[0]
    emb = emb_ref[...]
    C = emb.shape[0]
    P = ze.shape[1]
    D = ze.shape[0]

    @pl.when(b == 0)
    def _():
        enorm_ref[...] = jnp.sum(emb * emb, axis=1, keepdims=True)

    scores = jnp.dot(emb, ze, preferred_element_type=jnp.float32)
    fnorm = jnp.sum(ze * ze, axis=0, keepdims=True)
    dist = (fnorm + enorm_ref[...]) - 2.0 * scores
    idx = jnp.argmin(dist, axis=0)
    onehot = (jax.lax.broadcasted_iota(jnp.int32, (C, P), 0)
              == idx[None, :]).astype(jnp.float32)
    zq = jax.lax.dot_general(emb, onehot, (((0,), (0,)), ((), ())),
                             preferred_element_type=jnp.float32)
    zq_ref[0] = zq
    idx_ref[0] = idx.reshape(1, P).astype(jnp.int32)
    diff = ze - zq
    part = jnp.sum(diff * diff).reshape(1, 1)

    @pl.when(b == 0)
    def _():
        loss_ref[...] = part

    @pl.when(b != 0)
    def _():
        loss_ref[...] += part

    @pl.when(b == nb - 1)
    def _():
        loss_ref[...] = loss_ref[...] / (nb * D * P)


def kernel(z_e, embed):
    B, D, H, W = z_e.shape
    P = H * W
    C = embed.shape[0]
    ze = z_e.reshape(B, D, P)

    zq, idx, loss = pl.pallas_call(
        _vq_body,
        grid=(B,),
        in_specs=[
            pl.BlockSpec((1, D, P), lambda b: (b, 0, 0)),
            pl.BlockSpec((C, D), lambda b: (0, 0)),
        ],
        out_specs=[
            pl.BlockSpec((1, D, P), lambda b: (b, 0, 0)),
            pl.BlockSpec((1, 1, P), lambda b: (b, 0, 0)),
            pl.BlockSpec((1, 1), lambda b: (0, 0)),
        ],
        out_shape=[
            jax.ShapeDtypeStruct((B, D, P), jnp.float32),
            jax.ShapeDtypeStruct((B, 1, P), jnp.int32),
            jax.ShapeDtypeStruct((1, 1), jnp.float32),
        ],
        scratch_shapes=[pltpu.VMEM((C, 1), jnp.float32)],
    )(ze, embed)

    z_q_st = zq.reshape(B, D, H, W)
    commitment_loss = loss.reshape(())
    indices_out = idx.reshape(B, H, W)
    return (z_q_st, commitment_loss, indices_out)
